# Initial kernel scaffold; baseline (speedup 1.0000x reference)
import jax
import jax.numpy as jnp
from jax.experimental import pallas as pl


def kernel(output, target):
    raise NotImplementedError("write your pallas kernel here")


# speedup vs baseline: 1.0857x; 1.0857x over previous
import jax
import jax.numpy as jnp
from jax import lax
from jax.experimental import pallas as pl
from jax.experimental.pallas import tpu as pltpu
from jax.experimental.pallas import tpu_sc as plsc

_N_ROWS = 1024
_N_COLS = 100000
_TOPK = 5

_NC = 2
_NS = 16
_NW = _NC * _NS
_RPW = _N_ROWS // _NW


def _sc_gather_body(xflat_hbm, tgt_hbm, t_hbm, tgt_v, idx_v, val_v, sem):
    wid = lax.axis_index("s") * _NC + lax.axis_index("c")
    base = wid * _RPW
    pltpu.sync_copy(tgt_hbm.at[pl.ds(base, _RPW)], tgt_v)
    for k in range(_RPW // 16):
        tv = tgt_v[pl.ds(k * 16, 16)]
        rows = base + k * 16 + lax.broadcasted_iota(jnp.int32, (16,), 0)
        idx_v[pl.ds(k * 16, 16)] = rows * _N_COLS + tv
    pltpu.async_copy(xflat_hbm.at[idx_v], val_v, sem).wait()
    pltpu.sync_copy(val_v, t_hbm.at[pl.ds(base, _RPW)])


def _make_sc_gather():
    return pl.kernel(
        _sc_gather_body,
        out_type=jax.ShapeDtypeStruct((_N_ROWS,), jnp.float32),
        mesh=plsc.VectorSubcoreMesh(
            core_axis_name="c", subcore_axis_name="s",
            num_cores=_NC, num_subcores=_NS,
        ),
        scratch_types=[
            pltpu.VMEM((_RPW,), jnp.int32),
            pltpu.VMEM((_RPW,), jnp.int32),
            pltpu.VMEM((_RPW,), jnp.float32),
            pltpu.SemaphoreType.DMA,
        ],
    )

_RB = 256
_CB = 2048
_NCB = pl.cdiv(_N_COLS, _CB)


def _count_body(x_ref, t_ref, tgt_ref, out_ref, acc_ref):
    r = pl.program_id(0)
    c = pl.program_id(1)

    @pl.when(c == 0)
    def _():
        acc_ref[...] = jnp.zeros_like(acc_ref)

    x = x_ref[...]
    t = t_ref[...]
    tgt = tgt_ref[...]
    col = c * _CB + lax.broadcasted_iota(jnp.int32, (_RB, _CB), 1)
    ahead = ((x > t) | ((x == t) & (col < tgt))) & (col < _N_COLS)
    acc_ref[...] += jnp.sum(ahead.astype(jnp.int32), axis=1, keepdims=True)

    @pl.when(c == _NCB - 1)
    def _():
        @pl.when(r == 0)
        def _():
            out_ref[...] = jnp.zeros_like(out_ref)

        correct = (acc_ref[...] < _TOPK).astype(jnp.float32)
        out_ref[...] += jnp.sum(correct).reshape(1, 1) * (100.0 / _N_ROWS)


_count = pl.pallas_call(
    _count_body,
    grid=(_N_ROWS // _RB, _NCB),
    in_specs=[
        pl.BlockSpec((_RB, _CB), lambda r, c: (r, c)),
        pl.BlockSpec((_RB, 1), lambda r, c: (r, 0)),
        pl.BlockSpec((_RB, 1), lambda r, c: (r, 0)),
    ],
    out_specs=pl.BlockSpec((1, 1), lambda r, c: (0, 0)),
    out_shape=jax.ShapeDtypeStruct((1, 1), jnp.float32),
    scratch_shapes=[pltpu.VMEM((_RB, 1), jnp.int32)],
)


def kernel(output, target):
    tgt = target.astype(jnp.int32)
    t = _make_sc_gather()(output.reshape(-1), tgt)
    res = _count(output, t.reshape(_N_ROWS, 1), tgt.reshape(_N_ROWS, 1))
    return res.reshape(1)


# speedup vs baseline: 2.0661x; 1.9030x over previous
import jax
import jax.numpy as jnp
from jax import lax
from jax.experimental import pallas as pl
from jax.experimental.pallas import tpu as pltpu
from jax.experimental.pallas import tpu_sc as plsc

_N_ROWS = 1024
_N_COLS = 100000
_TOPK = 5

_NC = 2
_NS = 16
_NW = _NC * _NS
_RPW = _N_ROWS // _NW


def _sc_gather_body(xflat_hbm, tgt_hbm, t_hbm, tgt_v, idx_v, val_v, sem):
    wid = lax.axis_index("s") * _NC + lax.axis_index("c")
    base = wid * _RPW
    pltpu.sync_copy(tgt_hbm.at[pl.ds(base, _RPW)], tgt_v)
    for k in range(_RPW // 16):
        tv = tgt_v[pl.ds(k * 16, 16)]
        rows = base + k * 16 + lax.broadcasted_iota(jnp.int32, (16,), 0)
        idx_v[pl.ds(k * 16, 16)] = rows * _N_COLS + tv
    pltpu.async_copy(xflat_hbm.at[idx_v], val_v, sem).wait()
    pltpu.sync_copy(val_v, t_hbm.at[pl.ds(base, _RPW)])


def _make_sc_gather():
    return pl.kernel(
        _sc_gather_body,
        out_type=jax.ShapeDtypeStruct((_N_ROWS,), jnp.float32),
        mesh=plsc.VectorSubcoreMesh(
            core_axis_name="c", subcore_axis_name="s",
            num_cores=_NC, num_subcores=_NS,
        ),
        scratch_types=[
            pltpu.VMEM((_RPW,), jnp.int32),
            pltpu.VMEM((_RPW,), jnp.int32),
            pltpu.VMEM((_RPW,), jnp.float32),
            pltpu.SemaphoreType.DMA,
        ],
    )

_RB = 256
_CB = 2048
_NCB = pl.cdiv(_N_COLS, _CB)


def _count_body(x_ref, t_ref, tgt_ref, out_ref, acc_ref):
    r = pl.program_id(0)
    c = pl.program_id(1)

    @pl.when(c == 0)
    def _():
        acc_ref[...] = jnp.zeros_like(acc_ref)

    x = x_ref[...]
    t = t_ref[...]
    tgt = tgt_ref[...]
    col = c * _CB + lax.broadcasted_iota(jnp.int32, (_RB, _CB), 1)
    ahead = ((x > t) | ((x == t) & (col < tgt))) & (col < _N_COLS)
    acc_ref[...] += jnp.sum(ahead.astype(jnp.int32), axis=1, keepdims=True)

    @pl.when(c == _NCB - 1)
    def _():
        @pl.when(r == 0)
        def _():
            out_ref[...] = jnp.zeros_like(out_ref)

        correct = (acc_ref[...] < _TOPK).astype(jnp.float32)
        out_ref[...] += jnp.sum(correct).reshape(1, 1) * (100.0 / _N_ROWS)


_count = pl.pallas_call(
    _count_body,
    grid=(_N_ROWS // _RB, _NCB),
    in_specs=[
        pl.BlockSpec((_RB, _CB), lambda r, c: (r, c)),
        pl.BlockSpec((_RB, 1), lambda r, c: (r, 0)),
        pl.BlockSpec((_RB, 1), lambda r, c: (r, 0)),
    ],
    out_specs=pl.BlockSpec((1, 1), lambda r, c: (0, 0)),
    out_shape=jax.ShapeDtypeStruct((1, 1), jnp.float32),
    scratch_shapes=[pltpu.VMEM((_RB, 1), jnp.int32)],
)


def kernel(output, target):
    tgt = target.astype(jnp.int32)
    t = jnp.take_along_axis(output, tgt[:, None], axis=1).reshape(-1)
    res = _count(output, t.reshape(_N_ROWS, 1), tgt.reshape(_N_ROWS, 1))
    return res.reshape(1)


# speedup vs baseline: 2.3917x; 1.1576x over previous
import jax
import jax.numpy as jnp
from jax import lax
from jax.experimental import pallas as pl
from jax.experimental.pallas import tpu as pltpu
from jax.experimental.pallas import tpu_sc as plsc

_N_ROWS = 1024
_N_COLS = 100000
_TOPK = 5

_NC = 2
_NS = 16
_NW = _NC * _NS
_RPW = _N_ROWS // _NW


def _sc_gather_body(xflat_hbm, tgt_hbm, t_hbm, tgt_v, idx_v, val_v, sem):
    wid = lax.axis_index("s") * _NC + lax.axis_index("c")
    base = wid * _RPW
    pltpu.sync_copy(tgt_hbm.at[pl.ds(base, _RPW)], tgt_v)
    for k in range(_RPW // 16):
        tv = tgt_v[pl.ds(k * 16, 16)]
        rows = base + k * 16 + lax.broadcasted_iota(jnp.int32, (16,), 0)
        idx_v[pl.ds(k * 16, 16)] = rows * _N_COLS + tv
    pltpu.async_copy(xflat_hbm.at[idx_v], val_v, sem).wait()
    pltpu.sync_copy(val_v, t_hbm.at[pl.ds(base, _RPW)])


def _make_sc_gather():
    return pl.kernel(
        _sc_gather_body,
        out_type=jax.ShapeDtypeStruct((_N_ROWS,), jnp.float32),
        mesh=plsc.VectorSubcoreMesh(
            core_axis_name="c", subcore_axis_name="s",
            num_cores=_NC, num_subcores=_NS,
        ),
        scratch_types=[
            pltpu.VMEM((_RPW,), jnp.int32),
            pltpu.VMEM((_RPW,), jnp.int32),
            pltpu.VMEM((_RPW,), jnp.float32),
            pltpu.SemaphoreType.DMA,
        ],
    )

_RB = 1024
_CB = 2048
_NCB = pl.cdiv(_N_COLS, _CB)
_TAIL = _N_COLS - (_NCB - 1) * _CB


def _count_body(x_ref, t_ref, tgt_ref, out_ref, acc_ref, ones_ref):
    c = pl.program_id(0)

    @pl.when(c == 0)
    def _():
        acc_ref[...] = jnp.zeros_like(acc_ref)
        ones_ref[...] = jnp.ones_like(ones_ref)

    x = x_ref[...]
    t = t_ref[...]
    rel = tgt_ref[...] - c * _CB
    li = lax.broadcasted_iota(jnp.int32, (_RB, _CB), 1)
    limit = jnp.where(c == _NCB - 1, _TAIL, _CB)
    ahead = ((x > t) | ((x == t) & (li < rel))) & (li < limit)
    cnt = jnp.where(ahead, 1.0, 0.0)
    acc_ref[...] += jax.lax.dot_general(
        cnt, ones_ref[...], (((1,), (0,)), ((), ())),
        preferred_element_type=jnp.float32)

    @pl.when(c == _NCB - 1)
    def _():
        correct = (acc_ref[...] < float(_TOPK)).astype(jnp.float32)
        out_ref[...] = jnp.sum(correct).reshape(1, 1) * (100.0 / _N_ROWS)


_count = pl.pallas_call(
    _count_body,
    grid=(_NCB,),
    in_specs=[
        pl.BlockSpec((_RB, _CB), lambda c: (0, c)),
        pl.BlockSpec((_RB, 1), lambda c: (0, 0)),
        pl.BlockSpec((_RB, 1), lambda c: (0, 0)),
    ],
    out_specs=pl.BlockSpec((1, 1), lambda c: (0, 0)),
    out_shape=jax.ShapeDtypeStruct((1, 1), jnp.float32),
    scratch_shapes=[
        pltpu.VMEM((_RB, 1), jnp.float32),
        pltpu.VMEM((_CB, 1), jnp.float32),
    ],
)


def kernel(output, target):
    tgt = target.astype(jnp.int32)
    t = jnp.take_along_axis(output, tgt[:, None], axis=1).reshape(-1)
    res = _count(output, t.reshape(_N_ROWS, 1), tgt.reshape(_N_ROWS, 1))
    return res.reshape(1)
